# baseline (speedup 1.0000x reference)
import functools

import jax
import jax.numpy as jnp
from jax.experimental import pallas as pl
from jax.experimental.pallas import tpu as pltpu

BLOCK_R = 256
WINDOW = 64


def _dotT(a, w):
    return jax.lax.dot_general(
        a, w, (((1,), (1,)), ((), ())), preferred_element_type=jnp.float32
    )


def _fused_body(emb_ref, prev_ref, ww_ref, bw_ref, wq_ref, bq_ref,
                wo_ref, bo_ref, out_ref):
    i = pl.program_id(1)
    base = i * BLOCK_R
    scale = ww_ref.shape[0] ** -0.5

    e = emb_ref[0]
    ep = prev_ref[0]

    q = _dotT(e, wq_ref[...]) + bq_ref[...]
    k_cur = _dotT(e, ww_ref[...]) + bw_ref[...]
    k_prev = _dotT(ep, ww_ref[...]) + bw_ref[...]
    keys = jnp.concatenate([k_prev, k_cur], axis=0)
    kb = keys.astype(jnp.bfloat16)

    sim = _dotT(q.astype(jnp.bfloat16), kb) * scale

    rows = jax.lax.broadcasted_iota(jnp.int32, sim.shape, 0)
    cols = jax.lax.broadcasted_iota(jnp.int32, sim.shape, 1)
    band = (cols >= rows) & (cols <= rows + WINDOW - 1)
    nonneg = cols + base >= WINDOW
    mask = band & nonneg
    sim = jnp.where(mask, sim, -1e30)

    m = jnp.max(sim, axis=1, keepdims=True)
    p = jnp.exp(sim - m)
    denom = jnp.sum(p, axis=1, keepdims=True)
    attn = p / denom

    retrieved = jax.lax.dot_general(
        attn.astype(jnp.bfloat16), kb, (((1,), (0,)), ((), ())),
        preferred_element_type=jnp.float32,
    )

    out = _dotT(retrieved.astype(jnp.bfloat16), wo_ref[...]) + bo_ref[...]
    t0 = jax.lax.broadcasted_iota(jnp.int32, out.shape, 0) + base
    out = jnp.where(t0 > 0, out, 0.0)
    out_ref[0] = out


def kernel(embeddings, W_write, b_write, W_read_q, b_read_q, W_out, b_out):
    B, T, E = embeddings.shape
    D = W_write.shape[0]
    R, W = BLOCK_R, WINDOW
    n_blk = T // R

    grid = (B, n_blk)
    out = pl.pallas_call(
        _fused_body,
        grid=grid,
        in_specs=[
            pl.BlockSpec((1, R, E), lambda b, i: (b, i, 0)),
            pl.BlockSpec((1, W, E), lambda b, i: (b, jnp.maximum(i * (R // W) - 1, 0), 0)),
            pl.BlockSpec((D, E), lambda b, i: (0, 0)),
            pl.BlockSpec((1, D), lambda b, i: (0, 0)),
            pl.BlockSpec((D, E), lambda b, i: (0, 0)),
            pl.BlockSpec((1, D), lambda b, i: (0, 0)),
            pl.BlockSpec((E, D), lambda b, i: (0, 0)),
            pl.BlockSpec((1, E), lambda b, i: (0, 0)),
        ],
        out_specs=pl.BlockSpec((1, R, E), lambda b, i: (b, i, 0)),
        out_shape=jax.ShapeDtypeStruct((B, T, E), jnp.float32),
    )(
        emb16 := embeddings.astype(jnp.bfloat16),
        emb16,
        W_write.astype(jnp.bfloat16),
        b_write.reshape(1, D),
        W_read_q.astype(jnp.bfloat16),
        b_read_q.reshape(1, D),
        W_out.astype(jnp.bfloat16),
        b_out.reshape(1, E),
    )
    return out


# speedup vs baseline: 1.8906x; 1.8906x over previous
import functools

import jax
import jax.numpy as jnp
from jax.experimental import pallas as pl
from jax.experimental.pallas import tpu as pltpu

BLOCK_R = 512
WINDOW = 64


def _dotT(a, w):
    return jax.lax.dot_general(
        a, w, (((1,), (1,)), ((), ())), preferred_element_type=jnp.float32
    )


def _fused_body(emb_ref, prev_ref, ww_ref, bw_ref, wq_ref, bq_ref,
                wo_ref, bo_ref, out_ref):
    i = pl.program_id(1)
    base = i * BLOCK_R
    scale = ww_ref.shape[0] ** -0.5

    e = emb_ref[0]
    ep = prev_ref[0]

    q = _dotT(e, wq_ref[...]) + bq_ref[...]
    k_cur = _dotT(e, ww_ref[...]) + bw_ref[...]
    k_prev = _dotT(ep, ww_ref[...]) + bw_ref[...]
    keys = jnp.concatenate([k_prev, k_cur], axis=0)

    sim = _dotT(q, keys) * scale

    rows = jax.lax.broadcasted_iota(jnp.int32, sim.shape, 0)
    cols = jax.lax.broadcasted_iota(jnp.int32, sim.shape, 1)
    band = (cols >= rows) & (cols <= rows + WINDOW - 1)
    nonneg = cols + base >= WINDOW
    mask = band & nonneg
    sim = jnp.where(mask, sim, -1e30)

    m = jnp.max(sim, axis=1, keepdims=True)
    p = jnp.exp(sim - m)
    denom = jnp.sum(p, axis=1, keepdims=True)
    attn = p / denom

    retrieved = jax.lax.dot_general(
        attn, keys, (((1,), (0,)), ((), ())), preferred_element_type=jnp.float32
    )

    out = _dotT(retrieved, wo_ref[...]) + bo_ref[...]
    t0 = jax.lax.broadcasted_iota(jnp.int32, out.shape, 0) + base
    out = jnp.where(t0 > 0, out, 0.0)
    out_ref[0] = out


def kernel(embeddings, W_write, b_write, W_read_q, b_read_q, W_out, b_out):
    B, T, E = embeddings.shape
    D = W_write.shape[0]
    R, W = BLOCK_R, WINDOW
    n_blk = T // R

    grid = (B, n_blk)
    out = pl.pallas_call(
        _fused_body,
        grid=grid,
        in_specs=[
            pl.BlockSpec((1, R, E), lambda b, i: (b, i, 0)),
            pl.BlockSpec((1, W, E), lambda b, i: (b, jnp.maximum(i * (R // W) - 1, 0), 0)),
            pl.BlockSpec((D, E), lambda b, i: (0, 0)),
            pl.BlockSpec((1, D), lambda b, i: (0, 0)),
            pl.BlockSpec((D, E), lambda b, i: (0, 0)),
            pl.BlockSpec((1, D), lambda b, i: (0, 0)),
            pl.BlockSpec((E, D), lambda b, i: (0, 0)),
            pl.BlockSpec((1, E), lambda b, i: (0, 0)),
        ],
        out_specs=pl.BlockSpec((1, R, E), lambda b, i: (b, i, 0)),
        out_shape=jax.ShapeDtypeStruct((B, T, E), jnp.float32),
    )(
        embeddings,
        embeddings,
        W_write,
        b_write.reshape(1, D),
        W_read_q,
        b_read_q.reshape(1, D),
        W_out,
        b_out.reshape(1, E),
    )
    return out
